# baseline (speedup 1.0000x reference)
import functools

import jax
import jax.numpy as jnp
from jax import lax
from jax.experimental import pallas as pl
from jax.experimental.pallas import tpu as pltpu
from jax.experimental.pallas import tpu_sc as plsc

N = 10000
DIN = 128
EMB = 64
REG = 10240
TOT = 2 * REG
NC, NS = 2, 16
ROWS_PER_TILE = TOT // NS
LANE = 128
E_TOT = 320000 + 160000
CHUNKS_PER_TILE = 120
QC = CHUNKS_PER_TILE // 4
TOT_CHUNKS = NC * NS * CHUNKS_PER_TILE
E_PAD = TOT_CHUNKS * LANE
DEG_W = 16
SCALE = EMB ** (-0.5)


def _seg_body(y, idx2, zer64, agg_out, idx_v, rows0, rows1, acc, sem0, sem1,
              deg=None):
    cid = lax.axis_index("c")
    sid = lax.axis_index("s")
    w = cid * NS + sid
    c2 = 2 * CHUNKS_PER_TILE
    row0 = sid * ROWS_PER_TILE
    pltpu.sync_copy(zer64, acc.at[pl.ds(row0, ROWS_PER_TILE)])
    if deg is not None:
        zer16, ones16, deg_out, ones_v, accd = deg
        pltpu.sync_copy(zer16, accd.at[pl.ds(row0, ROWS_PER_TILE)])
        pltpu.sync_copy(ones16, ones_v)
    plsc.subcore_barrier()

    def body2(jj, carry):
        j2 = jj * 4
        pltpu.async_copy(y.at[idx_v.at[j2 + 2]], rows1, sem1)
        pltpu.make_async_copy(y.at[idx_v.at[j2]], rows0, sem0).wait()
        pltpu.sync_copy(rows0, acc.at[idx_v.at[j2 + 1]], add=True)
        if deg is not None:
            pltpu.sync_copy(ones_v, accd.at[idx_v.at[j2 + 1]], add=True)

        @pl.when(jj * 2 + 2 < QC)
        def _():
            pltpu.async_copy(y.at[idx_v.at[j2 + 4]], rows0, sem0)

        pltpu.make_async_copy(y.at[idx_v.at[j2 + 2]], rows1, sem1).wait()
        pltpu.sync_copy(rows1, acc.at[idx_v.at[j2 + 3]], add=True)
        if deg is not None:
            pltpu.sync_copy(ones_v, accd.at[idx_v.at[j2 + 3]], add=True)
        return carry

    for h in range(CHUNKS_PER_TILE // QC):
        pltpu.sync_copy(idx2.at[pl.ds(w * c2 + h * 2 * QC, 2 * QC)], idx_v)
        pltpu.async_copy(y.at[idx_v.at[0]], rows0, sem0)
        lax.fori_loop(0, QC // 2, body2, 0)
    plsc.subcore_barrier()
    pltpu.sync_copy(acc.at[pl.ds(row0, ROWS_PER_TILE)],
                    agg_out.at[cid, pl.ds(row0, ROWS_PER_TILE)])
    if deg is not None:
        pltpu.sync_copy(accd.at[pl.ds(row0, ROWS_PER_TILE)],
                        deg_out.at[cid, pl.ds(row0, ROWS_PER_TILE)])


def _make_seg_sum(with_deg):
    mesh = plsc.VectorSubcoreMesh(core_axis_name="c", subcore_axis_name="s")
    outs = [jax.ShapeDtypeStruct((NC, TOT, EMB), jnp.float32)]
    scratch = [
        pltpu.VMEM((2 * QC, LANE), jnp.int32),
        pltpu.VMEM((LANE, EMB), jnp.float32),
        pltpu.VMEM((LANE, EMB), jnp.float32),
        pltpu.VMEM_SHARED((TOT, EMB), jnp.float32),
        pltpu.SemaphoreType.DMA,
        pltpu.SemaphoreType.DMA,
    ]
    if with_deg:
        outs.append(jax.ShapeDtypeStruct((NC, TOT, DEG_W), jnp.float32))
        scratch += [
            pltpu.VMEM((LANE, DEG_W), jnp.float32),
            pltpu.VMEM_SHARED((TOT, DEG_W), jnp.float32),
        ]

        def body(y, idx2, zer64, zer16, ones16, agg_out, deg_out,
                 idx_v, rows0, rows1, acc, sem0, sem1, ones_v, accd):
            _seg_body(y, idx2, zer64, agg_out, idx_v, rows0, rows1, acc,
                      sem0, sem1,
                      deg=(zer16, ones16, deg_out, ones_v, accd))
    else:
        def body(y, idx2, zer64, agg_out,
                 idx_v, rows0, rows1, acc, sem0, sem1):
            _seg_body(y, idx2, zer64, agg_out, idx_v, rows0, rows1, acc,
                      sem0, sem1, deg=None)

    return pl.kernel(body, out_type=tuple(outs) if with_deg else outs[0],
                     mesh=mesh, scratch_types=scratch,
                     compiler_params=pltpu.CompilerParams(
                         use_tc_tiling_on_sc=False))


@functools.cache
def _get_seg_sum(with_deg):
    return _make_seg_sum(with_deg)


def _seg_sum_deg(y, idx2, zer64, zer16, ones16):
    return _get_seg_sum(True)(y, idx2, zer64, zer16, ones16)


def _seg_sum(y, idx2, zer64):
    return _get_seg_sum(False)(y, idx2, zer64)


def _mm_dual_body(x_ref, wl_ref, wp_ref, o_ref):
    w = jnp.where(pl.program_id(0) < NS, wl_ref[...], wp_ref[...])
    o_ref[...] = jnp.dot(x_ref[...], w, preferred_element_type=jnp.float32)


def _mm_dual(x, wl, wp):
    blk = TOT // 32
    return pl.pallas_call(
        _mm_dual_body,
        grid=(32,),
        in_specs=[
            pl.BlockSpec((blk, DIN), lambda i: (i, 0)),
            pl.BlockSpec((DIN, EMB), lambda i: (0, 0)),
            pl.BlockSpec((DIN, EMB), lambda i: (0, 0)),
        ],
        out_specs=pl.BlockSpec((blk, EMB), lambda i: (i, 0)),
        out_shape=jax.ShapeDtypeStruct((TOT, EMB), jnp.float32),
    )(x, wl, wp)


def _combine_body(parts_ref, deg_ref, bli_ref, bpi_ref, wl_ref, wp_ref,
                  blo_ref, bpo_ref, o_ref):
    sel = pl.program_id(0) < NS
    p = parts_ref[0] + parts_ref[1]
    deg = deg_ref[0, :, 0:1] + deg_ref[1, :, 0:1]
    b_in = jnp.where(sel, bli_ref[...], bpi_ref[...])
    w = jnp.where(sel, wl_ref[...], wp_ref[...])
    b_out = jnp.where(sel, blo_ref[...], bpo_ref[...])
    h = jnp.maximum(p / jnp.maximum(deg, 1.0) + b_in, 0.0)
    o_ref[...] = jnp.dot(h, w, preferred_element_type=jnp.float32) + b_out


def _combine(parts, degp, bl_in, bp_in, wl, wp, bl_out, bp_out):
    blk = TOT // 32
    b2 = lambda b: b.reshape(1, EMB)
    return pl.pallas_call(
        _combine_body,
        grid=(32,),
        in_specs=[
            pl.BlockSpec((NC, blk, EMB), lambda i: (0, i, 0)),
            pl.BlockSpec((NC, blk, DEG_W), lambda i: (0, i, 0)),
            pl.BlockSpec((1, EMB), lambda i: (0, 0)),
            pl.BlockSpec((1, EMB), lambda i: (0, 0)),
            pl.BlockSpec((EMB, EMB), lambda i: (0, 0)),
            pl.BlockSpec((EMB, EMB), lambda i: (0, 0)),
            pl.BlockSpec((1, EMB), lambda i: (0, 0)),
            pl.BlockSpec((1, EMB), lambda i: (0, 0)),
        ],
        out_specs=pl.BlockSpec((blk, EMB), lambda i: (i, 0)),
        out_shape=jax.ShapeDtypeStruct((TOT, EMB), jnp.float32),
    )(parts, degp, b2(bl_in), b2(bp_in), wl, wp, b2(bl_out), b2(bp_out))


def _scores_body(a_ref, b_ref, o_ref):
    s = lax.dot_general(a_ref[...], b_ref[...], (((1,), (1,)), ((), ())),
                        preferred_element_type=jnp.float32)
    o_ref[...] = 1.0 / (1.0 + jnp.exp(-s * SCALE))


def _scores(ab):
    bm, bn = 1000, 512
    return pl.pallas_call(
        _scores_body,
        grid=(N // bm, -(-N // bn)),
        in_specs=[
            pl.BlockSpec((bm, EMB), lambda i, j: (i, 0)),
            pl.BlockSpec((bn, EMB), lambda i, j: (j + REG // bn, 0)),
        ],
        out_specs=pl.BlockSpec((bm, bn), lambda i, j: (i, j)),
        out_shape=jax.ShapeDtypeStruct((N, N), jnp.float32),
    )(ab, ab)


def kernel(logic_x, phys_x, Wl1, bl1, Wl2, bl2, Wp1, bp1, Wp2, bp2,
           Was, bas, Wap, bap, logic_edge_index, phys_edge_index):
    f32 = jnp.float32
    pad_rows = jnp.zeros((REG - N, DIN), f32)
    cat_x = jnp.concatenate([logic_x, pad_rows, phys_x, pad_rows], axis=0)

    src = jnp.concatenate([
        logic_edge_index[0].astype(jnp.int32),
        phys_edge_index[0].astype(jnp.int32) + REG,
        jnp.zeros((E_PAD - E_TOT,), jnp.int32),
    ])
    dst = jnp.concatenate([
        logic_edge_index[1].astype(jnp.int32),
        phys_edge_index[1].astype(jnp.int32) + REG,
        N + (jnp.arange(E_PAD - E_TOT, dtype=jnp.int32) % (REG - N)),
    ])
    nw = NC * NS
    idx2 = jnp.stack([src.reshape(CHUNKS_PER_TILE, nw, LANE),
                      dst.reshape(CHUNKS_PER_TILE, nw, LANE)],
                     axis=2).transpose(1, 0, 2, 3).reshape(
                         nw * CHUNKS_PER_TILE * 2, LANE)
    zer64 = jnp.zeros((ROWS_PER_TILE, EMB), f32)
    zer16 = jnp.zeros((ROWS_PER_TILE, DEG_W), f32)
    ones16 = jnp.ones((LANE, DEG_W), f32)
    zb = jnp.zeros((EMB,), f32)

    y1 = _mm_dual(cat_x, Wl1, Wp1)
    agg1, degp = _seg_sum_deg(y1, idx2, zer64, zer16, ones16)
    y2 = _combine(agg1, degp, bl1, bp1, Wl2, Wp2, zb, zb)
    agg2 = _seg_sum(y2, idx2, zer64)
    ab = _combine(agg2, degp, bl2, bp2, Was, Wap, bas, bap)
    return _scores(ab)


# speedup vs baseline: 1.2972x; 1.2972x over previous
import functools

import jax
import jax.numpy as jnp
from jax import lax
from jax.experimental import pallas as pl
from jax.experimental.pallas import tpu as pltpu
from jax.experimental.pallas import tpu_sc as plsc

N = 10000
DIN = 128
EMB = 64
REG = 10240
TOT = 2 * REG
NC, NS = 2, 16
ROWS_PER_TILE = TOT // NS
LANE = 128
E_TOT = 320000 + 160000
CHUNKS_PER_TILE = 120
QC = CHUNKS_PER_TILE
TOT_CHUNKS = NC * NS * CHUNKS_PER_TILE
E_PAD = TOT_CHUNKS * LANE
DEG_W = 16
SCALE = EMB ** (-0.5)


def _seg_body(y, idx2, zer64, agg_out, idx_v, rows0, rows1, acc, sem0, sem1,
              deg=None):
    cid = lax.axis_index("c")
    sid = lax.axis_index("s")
    w = cid * NS + sid
    c2 = 2 * CHUNKS_PER_TILE
    row0 = sid * ROWS_PER_TILE
    pltpu.sync_copy(zer64, acc.at[pl.ds(row0, ROWS_PER_TILE)])
    if deg is not None:
        zer16, ones16, deg_out, ones_v, accd = deg
        pltpu.sync_copy(zer16, accd.at[pl.ds(row0, ROWS_PER_TILE)])
        pltpu.sync_copy(ones16, ones_v)
    plsc.subcore_barrier()

    def body2(jj, carry):
        j2 = jj * 4
        pltpu.async_copy(y.at[idx_v.at[j2 + 2]], rows1, sem1)
        pltpu.make_async_copy(y.at[idx_v.at[j2]], rows0, sem0).wait()
        pltpu.sync_copy(rows0, acc.at[idx_v.at[j2 + 1]], add=True)
        if deg is not None:
            pltpu.sync_copy(ones_v, accd.at[idx_v.at[j2 + 1]], add=True)

        @pl.when(jj * 2 + 2 < QC)
        def _():
            pltpu.async_copy(y.at[idx_v.at[j2 + 4]], rows0, sem0)

        pltpu.make_async_copy(y.at[idx_v.at[j2 + 2]], rows1, sem1).wait()
        pltpu.sync_copy(rows1, acc.at[idx_v.at[j2 + 3]], add=True)
        if deg is not None:
            pltpu.sync_copy(ones_v, accd.at[idx_v.at[j2 + 3]], add=True)
        return carry

    for h in range(CHUNKS_PER_TILE // QC):
        pltpu.sync_copy(idx2.at[pl.ds(w * c2 + h * 2 * QC, 2 * QC)], idx_v)
        pltpu.async_copy(y.at[idx_v.at[0]], rows0, sem0)
        lax.fori_loop(0, QC // 2, body2, 0)
    plsc.subcore_barrier()
    pltpu.sync_copy(acc.at[pl.ds(row0, ROWS_PER_TILE)],
                    agg_out.at[cid, pl.ds(row0, ROWS_PER_TILE)])
    if deg is not None:
        pltpu.sync_copy(accd.at[pl.ds(row0, ROWS_PER_TILE)],
                        deg_out.at[cid, pl.ds(row0, ROWS_PER_TILE)])


def _make_seg_sum(with_deg):
    mesh = plsc.VectorSubcoreMesh(core_axis_name="c", subcore_axis_name="s")
    outs = [jax.ShapeDtypeStruct((NC, TOT, EMB), jnp.bfloat16)]
    scratch = [
        pltpu.VMEM((2 * QC, LANE), jnp.int32),
        pltpu.VMEM((LANE, EMB), jnp.bfloat16),
        pltpu.VMEM((LANE, EMB), jnp.bfloat16),
        pltpu.VMEM_SHARED((TOT, EMB), jnp.bfloat16),
        pltpu.SemaphoreType.DMA,
        pltpu.SemaphoreType.DMA,
    ]
    if with_deg:
        outs.append(jax.ShapeDtypeStruct((NC, TOT, DEG_W), jnp.float32))
        scratch += [
            pltpu.VMEM((LANE, DEG_W), jnp.float32),
            pltpu.VMEM_SHARED((TOT, DEG_W), jnp.float32),
        ]

        def body(y, idx2, zer64, zer16, ones16, agg_out, deg_out,
                 idx_v, rows0, rows1, acc, sem0, sem1, ones_v, accd):
            _seg_body(y, idx2, zer64, agg_out, idx_v, rows0, rows1, acc,
                      sem0, sem1,
                      deg=(zer16, ones16, deg_out, ones_v, accd))
    else:
        def body(y, idx2, zer64, agg_out,
                 idx_v, rows0, rows1, acc, sem0, sem1):
            _seg_body(y, idx2, zer64, agg_out, idx_v, rows0, rows1, acc,
                      sem0, sem1, deg=None)

    return pl.kernel(body, out_type=tuple(outs) if with_deg else outs[0],
                     mesh=mesh, scratch_types=scratch,
                     compiler_params=pltpu.CompilerParams(
                         use_tc_tiling_on_sc=False))


@functools.cache
def _get_seg_sum(with_deg):
    return _make_seg_sum(with_deg)


def _seg_sum_deg(y, idx2, zer64, zer16, ones16):
    return _get_seg_sum(True)(y, idx2, zer64, zer16, ones16)


def _seg_sum(y, idx2, zer64):
    return _get_seg_sum(False)(y, idx2, zer64)


def _mm_dual_body(x_ref, wl_ref, wp_ref, o_ref):
    w = jnp.where(pl.program_id(0) < NS, wl_ref[...], wp_ref[...])
    o_ref[...] = jnp.dot(x_ref[...], w,
                         preferred_element_type=jnp.float32
                         ).astype(jnp.bfloat16)


def _mm_dual(x, wl, wp):
    blk = TOT // 32
    return pl.pallas_call(
        _mm_dual_body,
        grid=(32,),
        in_specs=[
            pl.BlockSpec((blk, DIN), lambda i: (i, 0)),
            pl.BlockSpec((DIN, EMB), lambda i: (0, 0)),
            pl.BlockSpec((DIN, EMB), lambda i: (0, 0)),
        ],
        out_specs=pl.BlockSpec((blk, EMB), lambda i: (i, 0)),
        out_shape=jax.ShapeDtypeStruct((TOT, EMB), jnp.bfloat16),
    )(x, wl, wp)


def _combine_body(parts_ref, deg_ref, bli_ref, bpi_ref, wl_ref, wp_ref,
                  blo_ref, bpo_ref, o_ref):
    sel = pl.program_id(0) < NS
    p = (parts_ref[0].astype(jnp.float32) + parts_ref[1].astype(jnp.float32))
    deg = deg_ref[0, :, 0:1] + deg_ref[1, :, 0:1]
    b_in = jnp.where(sel, bli_ref[...], bpi_ref[...])
    w = jnp.where(sel, wl_ref[...], wp_ref[...])
    b_out = jnp.where(sel, blo_ref[...], bpo_ref[...])
    h = jnp.maximum(p / jnp.maximum(deg, 1.0) + b_in, 0.0)
    o_ref[...] = (jnp.dot(h, w, preferred_element_type=jnp.float32)
                  + b_out).astype(o_ref.dtype)


def _combine(parts, degp, bl_in, bp_in, wl, wp, bl_out, bp_out, out_dtype):
    blk = TOT // 32
    b2 = lambda b: b.reshape(1, EMB)
    return pl.pallas_call(
        _combine_body,
        grid=(32,),
        in_specs=[
            pl.BlockSpec((NC, blk, EMB), lambda i: (0, i, 0)),
            pl.BlockSpec((NC, blk, DEG_W), lambda i: (0, i, 0)),
            pl.BlockSpec((1, EMB), lambda i: (0, 0)),
            pl.BlockSpec((1, EMB), lambda i: (0, 0)),
            pl.BlockSpec((EMB, EMB), lambda i: (0, 0)),
            pl.BlockSpec((EMB, EMB), lambda i: (0, 0)),
            pl.BlockSpec((1, EMB), lambda i: (0, 0)),
            pl.BlockSpec((1, EMB), lambda i: (0, 0)),
        ],
        out_specs=pl.BlockSpec((blk, EMB), lambda i: (i, 0)),
        out_shape=jax.ShapeDtypeStruct((TOT, EMB), out_dtype),
    )(parts, degp, b2(bl_in), b2(bp_in), wl, wp, b2(bl_out), b2(bp_out))


def _scores_body(a_ref, b_ref, o_ref):
    s = lax.dot_general(a_ref[...], b_ref[...], (((1,), (1,)), ((), ())),
                        preferred_element_type=jnp.float32)
    o_ref[...] = 1.0 / (1.0 + jnp.exp(-s * SCALE))


def _scores(ab):
    bm, bn = 1000, 512
    return pl.pallas_call(
        _scores_body,
        grid=(N // bm, -(-N // bn)),
        in_specs=[
            pl.BlockSpec((bm, EMB), lambda i, j: (i, 0)),
            pl.BlockSpec((bn, EMB), lambda i, j: (j + REG // bn, 0)),
        ],
        out_specs=pl.BlockSpec((bm, bn), lambda i, j: (i, j)),
        out_shape=jax.ShapeDtypeStruct((N, N), jnp.float32),
    )(ab, ab)


def kernel(logic_x, phys_x, Wl1, bl1, Wl2, bl2, Wp1, bp1, Wp2, bp2,
           Was, bas, Wap, bap, logic_edge_index, phys_edge_index):
    f32 = jnp.float32
    pad_rows = jnp.zeros((REG - N, DIN), f32)
    cat_x = jnp.concatenate([logic_x, pad_rows, phys_x, pad_rows], axis=0)

    src = jnp.concatenate([
        logic_edge_index[0].astype(jnp.int32),
        phys_edge_index[0].astype(jnp.int32) + REG,
        jnp.zeros((E_PAD - E_TOT,), jnp.int32),
    ])
    dst = jnp.concatenate([
        logic_edge_index[1].astype(jnp.int32),
        phys_edge_index[1].astype(jnp.int32) + REG,
        N + (jnp.arange(E_PAD - E_TOT, dtype=jnp.int32) % (REG - N)),
    ])
    nw = NC * NS
    idx2 = jnp.stack([src.reshape(CHUNKS_PER_TILE, nw, LANE),
                      dst.reshape(CHUNKS_PER_TILE, nw, LANE)],
                     axis=2).transpose(1, 0, 2, 3).reshape(
                         nw * CHUNKS_PER_TILE * 2, LANE)
    zer64 = jnp.zeros((ROWS_PER_TILE, EMB), jnp.bfloat16)
    zer16 = jnp.zeros((ROWS_PER_TILE, DEG_W), f32)
    ones16 = jnp.ones((LANE, DEG_W), f32)
    zb = jnp.zeros((EMB,), f32)

    y1 = _mm_dual(cat_x, Wl1, Wp1)
    agg1, degp = _seg_sum_deg(y1, idx2, zer64, zer16, ones16)
    y2 = _combine(agg1, degp, bl1, bp1, Wl2, Wp2, zb, zb, jnp.bfloat16)
    agg2 = _seg_sum(y2, idx2, zer64)
    ab = _combine(agg2, degp, bl2, bp2, Was, Wap, bas, bap, jnp.float32)
    return _scores(ab)


# speedup vs baseline: 1.3743x; 1.0594x over previous
import functools

import jax
import jax.numpy as jnp
from jax import lax
from jax.experimental import pallas as pl
from jax.experimental.pallas import tpu as pltpu
from jax.experimental.pallas import tpu_sc as plsc

N = 10000
DIN = 128
EMB = 64
REG = 10240
TOT = 2 * REG
NC, NS = 2, 16
ROWS_PER_TILE = TOT // NS
LANE = 128
E_TOT = 320000 + 160000
CHUNKS_PER_TILE = 120
QC = CHUNKS_PER_TILE
TOT_CHUNKS = NC * NS * CHUNKS_PER_TILE
E_PAD = TOT_CHUNKS * LANE
DEG_W = 16
SCALE = EMB ** (-0.5)


def _seg_body(y, idx2, zer64, agg_out, idx_v, rows, acc, semg, sems,
              deg=None):
    cid = lax.axis_index("c")
    sid = lax.axis_index("s")
    w = cid * NS + sid
    c2 = 2 * CHUNKS_PER_TILE
    C = CHUNKS_PER_TILE
    row0 = sid * ROWS_PER_TILE
    pltpu.sync_copy(zer64, acc.at[pl.ds(row0, ROWS_PER_TILE)])
    if deg is not None:
        zer16, ones16, deg_out, ones_v, accd = deg
        pltpu.sync_copy(zer16, accd.at[pl.ds(row0, ROWS_PER_TILE)])
        pltpu.sync_copy(ones16, ones_v)
    pltpu.sync_copy(idx2.at[pl.ds(w * c2, c2)], idx_v)
    plsc.subcore_barrier()

    def step(j, b, wait_scat, start_gather):
        bs = (b + 2) % 4
        if wait_scat:
            pltpu.make_async_copy(rows[bs], acc.at[idx_v.at[(j - 2) * 2 + 1]],
                                  sems[bs]).wait()
            if deg is not None:
                pltpu.make_async_copy(ones_v,
                                      accd.at[idx_v.at[(j - 2) * 2 + 1]],
                                      sems[bs]).wait()
        if start_gather:
            pltpu.async_copy(y.at[idx_v.at[(j + 2) * 2]], rows[bs], semg[bs])
        pltpu.make_async_copy(y.at[idx_v.at[j * 2]], rows[b], semg[b]).wait()
        pltpu.async_copy(rows[b], acc.at[idx_v.at[j * 2 + 1]], sems[b],
                         add=True)
        if deg is not None:
            pltpu.async_copy(ones_v, accd.at[idx_v.at[j * 2 + 1]], sems[b],
                             add=True)

    pltpu.async_copy(y.at[idx_v.at[0]], rows[0], semg[0])
    pltpu.async_copy(y.at[idx_v.at[2]], rows[1], semg[1])
    step(0, 0, False, True)
    step(1, 1, False, True)
    step(2, 2, True, True)
    step(3, 3, True, True)

    def body4(jj, carry):
        for u in range(4):
            step(jj * 4 + u, u, True, True)
        return carry

    lax.fori_loop(1, C // 4 - 1, body4, 0)
    step(C - 4, 0, True, True)
    step(C - 3, 1, True, True)
    step(C - 2, 2, True, False)
    step(C - 1, 3, True, False)
    for j in (C - 2, C - 1):
        b = j % 4
        pltpu.make_async_copy(rows[b], acc.at[idx_v.at[j * 2 + 1]],
                              sems[b]).wait()
        if deg is not None:
            pltpu.make_async_copy(ones_v, accd.at[idx_v.at[j * 2 + 1]],
                                  sems[b]).wait()
    plsc.subcore_barrier()
    pltpu.sync_copy(acc.at[pl.ds(row0, ROWS_PER_TILE)],
                    agg_out.at[cid, pl.ds(row0, ROWS_PER_TILE)])
    if deg is not None:
        pltpu.sync_copy(accd.at[pl.ds(row0, ROWS_PER_TILE)],
                        deg_out.at[cid, pl.ds(row0, ROWS_PER_TILE)])


def _make_seg_sum(with_deg):
    mesh = plsc.VectorSubcoreMesh(core_axis_name="c", subcore_axis_name="s")
    outs = [jax.ShapeDtypeStruct((NC, TOT, EMB), jnp.bfloat16)]
    scratch = [
        pltpu.VMEM((2 * QC, LANE), jnp.int32),
        pltpu.VMEM((LANE, EMB), jnp.bfloat16),
        pltpu.VMEM((LANE, EMB), jnp.bfloat16),
        pltpu.VMEM((LANE, EMB), jnp.bfloat16),
        pltpu.VMEM((LANE, EMB), jnp.bfloat16),
        pltpu.VMEM_SHARED((TOT, EMB), jnp.bfloat16),
        pltpu.SemaphoreType.DMA, pltpu.SemaphoreType.DMA,
        pltpu.SemaphoreType.DMA, pltpu.SemaphoreType.DMA,
        pltpu.SemaphoreType.DMA, pltpu.SemaphoreType.DMA,
        pltpu.SemaphoreType.DMA, pltpu.SemaphoreType.DMA,
    ]
    if with_deg:
        outs.append(jax.ShapeDtypeStruct((NC, TOT, DEG_W), jnp.float32))
        scratch += [
            pltpu.VMEM((LANE, DEG_W), jnp.float32),
            pltpu.VMEM_SHARED((TOT, DEG_W), jnp.float32),
        ]

        def body(y, idx2, zer64, zer16, ones16, agg_out, deg_out,
                 idx_v, r0, r1, r2, r3, acc, g0, g1, g2, g3, s0, s1, s2, s3,
                 ones_v, accd):
            _seg_body(y, idx2, zer64, agg_out, idx_v, [r0, r1, r2, r3], acc,
                      [g0, g1, g2, g3], [s0, s1, s2, s3],
                      deg=(zer16, ones16, deg_out, ones_v, accd))
    else:
        def body(y, idx2, zer64, agg_out,
                 idx_v, r0, r1, r2, r3, acc, g0, g1, g2, g3, s0, s1, s2, s3):
            _seg_body(y, idx2, zer64, agg_out, idx_v, [r0, r1, r2, r3], acc,
                      [g0, g1, g2, g3], [s0, s1, s2, s3], deg=None)

    return pl.kernel(body, out_type=tuple(outs) if with_deg else outs[0],
                     mesh=mesh, scratch_types=scratch,
                     compiler_params=pltpu.CompilerParams(
                         use_tc_tiling_on_sc=False))


@functools.cache
def _get_seg_sum(with_deg):
    return _make_seg_sum(with_deg)


def _seg_sum_deg(y, idx2, zer64, zer16, ones16):
    return _get_seg_sum(True)(y, idx2, zer64, zer16, ones16)


def _seg_sum(y, idx2, zer64):
    return _get_seg_sum(False)(y, idx2, zer64)


def _mm_dual_body(x_ref, wl_ref, wp_ref, o_ref):
    w = jnp.where(pl.program_id(0) < NS, wl_ref[...], wp_ref[...])
    o_ref[...] = jnp.dot(x_ref[...], w,
                         preferred_element_type=jnp.float32
                         ).astype(jnp.bfloat16)


def _mm_dual(x, wl, wp):
    blk = TOT // 32
    return pl.pallas_call(
        _mm_dual_body,
        grid=(32,),
        in_specs=[
            pl.BlockSpec((blk, DIN), lambda i: (i, 0)),
            pl.BlockSpec((DIN, EMB), lambda i: (0, 0)),
            pl.BlockSpec((DIN, EMB), lambda i: (0, 0)),
        ],
        out_specs=pl.BlockSpec((blk, EMB), lambda i: (i, 0)),
        out_shape=jax.ShapeDtypeStruct((TOT, EMB), jnp.bfloat16),
    )(x, wl, wp)


def _combine_body(parts_ref, deg_ref, bli_ref, bpi_ref, wl_ref, wp_ref,
                  blo_ref, bpo_ref, o_ref):
    sel = pl.program_id(0) < NS
    p = (parts_ref[0].astype(jnp.float32) + parts_ref[1].astype(jnp.float32))
    deg = deg_ref[0, :, 0:1] + deg_ref[1, :, 0:1]
    b_in = jnp.where(sel, bli_ref[...], bpi_ref[...])
    w = jnp.where(sel, wl_ref[...], wp_ref[...])
    b_out = jnp.where(sel, blo_ref[...], bpo_ref[...])
    h = jnp.maximum(p / jnp.maximum(deg, 1.0) + b_in, 0.0)
    o_ref[...] = (jnp.dot(h, w, preferred_element_type=jnp.float32)
                  + b_out).astype(o_ref.dtype)


def _combine(parts, degp, bl_in, bp_in, wl, wp, bl_out, bp_out, out_dtype):
    blk = TOT // 32
    b2 = lambda b: b.reshape(1, EMB)
    return pl.pallas_call(
        _combine_body,
        grid=(32,),
        in_specs=[
            pl.BlockSpec((NC, blk, EMB), lambda i: (0, i, 0)),
            pl.BlockSpec((NC, blk, DEG_W), lambda i: (0, i, 0)),
            pl.BlockSpec((1, EMB), lambda i: (0, 0)),
            pl.BlockSpec((1, EMB), lambda i: (0, 0)),
            pl.BlockSpec((EMB, EMB), lambda i: (0, 0)),
            pl.BlockSpec((EMB, EMB), lambda i: (0, 0)),
            pl.BlockSpec((1, EMB), lambda i: (0, 0)),
            pl.BlockSpec((1, EMB), lambda i: (0, 0)),
        ],
        out_specs=pl.BlockSpec((blk, EMB), lambda i: (i, 0)),
        out_shape=jax.ShapeDtypeStruct((TOT, EMB), out_dtype),
    )(parts, degp, b2(bl_in), b2(bp_in), wl, wp, b2(bl_out), b2(bp_out))


def _scores_body(a_ref, b_ref, o_ref):
    s = lax.dot_general(a_ref[...], b_ref[...], (((1,), (1,)), ((), ())),
                        preferred_element_type=jnp.float32)
    o_ref[...] = 1.0 / (1.0 + jnp.exp(-s * SCALE))


def _scores(ab):
    bm, bn = 1000, 512
    return pl.pallas_call(
        _scores_body,
        grid=(N // bm, -(-N // bn)),
        in_specs=[
            pl.BlockSpec((bm, EMB), lambda i, j: (i, 0)),
            pl.BlockSpec((bn, EMB), lambda i, j: (j + REG // bn, 0)),
        ],
        out_specs=pl.BlockSpec((bm, bn), lambda i, j: (i, j)),
        out_shape=jax.ShapeDtypeStruct((N, N), jnp.float32),
    )(ab, ab)


def kernel(logic_x, phys_x, Wl1, bl1, Wl2, bl2, Wp1, bp1, Wp2, bp2,
           Was, bas, Wap, bap, logic_edge_index, phys_edge_index):
    f32 = jnp.float32
    pad_rows = jnp.zeros((REG - N, DIN), f32)
    cat_x = jnp.concatenate([logic_x, pad_rows, phys_x, pad_rows], axis=0)

    src = jnp.concatenate([
        logic_edge_index[0].astype(jnp.int32),
        phys_edge_index[0].astype(jnp.int32) + REG,
        jnp.zeros((E_PAD - E_TOT,), jnp.int32),
    ])
    dst = jnp.concatenate([
        logic_edge_index[1].astype(jnp.int32),
        phys_edge_index[1].astype(jnp.int32) + REG,
        N + (jnp.arange(E_PAD - E_TOT, dtype=jnp.int32) % (REG - N)),
    ])
    nw = NC * NS
    idx2 = jnp.stack([src.reshape(CHUNKS_PER_TILE, nw, LANE),
                      dst.reshape(CHUNKS_PER_TILE, nw, LANE)],
                     axis=2).transpose(1, 0, 2, 3).reshape(
                         nw * CHUNKS_PER_TILE * 2, LANE)
    zer64 = jnp.zeros((ROWS_PER_TILE, EMB), jnp.bfloat16)
    zer16 = jnp.zeros((ROWS_PER_TILE, DEG_W), f32)
    ones16 = jnp.ones((LANE, DEG_W), f32)
    zb = jnp.zeros((EMB,), f32)

    y1 = _mm_dual(cat_x, Wl1, Wp1)
    agg1, degp = _seg_sum_deg(y1, idx2, zer64, zer16, ones16)
    y2 = _combine(agg1, degp, bl1, bp1, Wl2, Wp2, zb, zb, jnp.bfloat16)
    agg2 = _seg_sum(y2, idx2, zer64)
    ab = _combine(agg2, degp, bl2, bp2, Was, Wap, bas, bap, jnp.float32)
    return _scores(ab)


# speedup vs baseline: 1.5056x; 1.0955x over previous
import functools

import jax
import jax.numpy as jnp
from jax import lax
from jax.experimental import pallas as pl
from jax.experimental.pallas import tpu as pltpu
from jax.experimental.pallas import tpu_sc as plsc

N = 10000
DIN = 128
EMB = 64
REG = 10240
TOT = 2 * REG
NC, NS = 2, 16
ROWS_PER_TILE = TOT // NS
LANE = 128
E_TOT = 320000 + 160000
CHUNKS_PER_TILE = 120
QC = CHUNKS_PER_TILE
TOT_CHUNKS = NC * NS * CHUNKS_PER_TILE
E_PAD = TOT_CHUNKS * LANE
DEG_W = 16
SCALE = EMB ** (-0.5)


def _seg_body(y, idx2, zer64, agg_out, idx_v, rows, acc, semg, sems,
              deg=None):
    cid = lax.axis_index("c")
    sid = lax.axis_index("s")
    w = cid * NS + sid
    c2 = 2 * CHUNKS_PER_TILE
    C = CHUNKS_PER_TILE
    row0 = sid * ROWS_PER_TILE
    pltpu.sync_copy(zer64, acc.at[pl.ds(row0, ROWS_PER_TILE)])
    if deg is not None:
        zer16, ones16, deg_out, ones_v, accd = deg
        pltpu.sync_copy(zer16, accd.at[pl.ds(row0, ROWS_PER_TILE)])
        pltpu.sync_copy(ones16, ones_v)
    pltpu.sync_copy(idx2.at[pl.ds(w * c2, c2)], idx_v)
    plsc.subcore_barrier()

    def step(j, b, wait_scat, start_gather):
        bs = (b + 2) % 4
        if wait_scat:
            pltpu.make_async_copy(rows[bs], acc.at[idx_v.at[(j - 2) * 2 + 1]],
                                  sems[bs]).wait()
            if deg is not None:
                pltpu.make_async_copy(ones_v,
                                      accd.at[idx_v.at[(j - 2) * 2 + 1]],
                                      sems[bs]).wait()
        if start_gather:
            pltpu.async_copy(y.at[idx_v.at[(j + 2) * 2]], rows[bs], semg[bs])
        pltpu.make_async_copy(y.at[idx_v.at[j * 2]], rows[b], semg[b]).wait()
        pltpu.async_copy(rows[b], acc.at[idx_v.at[j * 2 + 1]], sems[b],
                         add=True)
        if deg is not None:
            pltpu.async_copy(ones_v, accd.at[idx_v.at[j * 2 + 1]], sems[b],
                             add=True)

    pltpu.async_copy(y.at[idx_v.at[0]], rows[0], semg[0])
    pltpu.async_copy(y.at[idx_v.at[2]], rows[1], semg[1])
    step(0, 0, False, True)
    step(1, 1, False, True)
    step(2, 2, True, True)
    step(3, 3, True, True)

    def body4(jj, carry):
        for u in range(4):
            step(jj * 4 + u, u, True, True)
        return carry

    lax.fori_loop(1, C // 4 - 1, body4, 0)
    step(C - 4, 0, True, True)
    step(C - 3, 1, True, True)
    step(C - 2, 2, True, False)
    step(C - 1, 3, True, False)
    for j in (C - 2, C - 1):
        b = j % 4
        pltpu.make_async_copy(rows[b], acc.at[idx_v.at[j * 2 + 1]],
                              sems[b]).wait()
        if deg is not None:
            pltpu.make_async_copy(ones_v, accd.at[idx_v.at[j * 2 + 1]],
                                  sems[b]).wait()
    plsc.subcore_barrier()
    pltpu.sync_copy(acc.at[pl.ds(row0, ROWS_PER_TILE)],
                    agg_out.at[cid, pl.ds(row0, ROWS_PER_TILE)])
    if deg is not None:
        pltpu.sync_copy(accd.at[pl.ds(row0, ROWS_PER_TILE)],
                        deg_out.at[cid, pl.ds(row0, ROWS_PER_TILE)])


def _make_seg_sum(with_deg):
    mesh = plsc.VectorSubcoreMesh(core_axis_name="c", subcore_axis_name="s")
    outs = [jax.ShapeDtypeStruct((NC, TOT, EMB), jnp.bfloat16)]
    scratch = [
        pltpu.VMEM((2 * QC, LANE), jnp.int32),
        pltpu.VMEM((LANE, EMB), jnp.bfloat16),
        pltpu.VMEM((LANE, EMB), jnp.bfloat16),
        pltpu.VMEM((LANE, EMB), jnp.bfloat16),
        pltpu.VMEM((LANE, EMB), jnp.bfloat16),
        pltpu.VMEM_SHARED((TOT, EMB), jnp.bfloat16),
        pltpu.SemaphoreType.DMA, pltpu.SemaphoreType.DMA,
        pltpu.SemaphoreType.DMA, pltpu.SemaphoreType.DMA,
        pltpu.SemaphoreType.DMA, pltpu.SemaphoreType.DMA,
        pltpu.SemaphoreType.DMA, pltpu.SemaphoreType.DMA,
    ]
    if with_deg:
        outs.append(jax.ShapeDtypeStruct((NC, TOT, DEG_W), jnp.float32))
        scratch += [
            pltpu.VMEM((LANE, DEG_W), jnp.float32),
            pltpu.VMEM_SHARED((TOT, DEG_W), jnp.float32),
        ]

        def body(y, idx2, zer64, zer16, ones16, agg_out, deg_out,
                 idx_v, r0, r1, r2, r3, acc, g0, g1, g2, g3, s0, s1, s2, s3,
                 ones_v, accd):
            _seg_body(y, idx2, zer64, agg_out, idx_v, [r0, r1, r2, r3], acc,
                      [g0, g1, g2, g3], [s0, s1, s2, s3],
                      deg=(zer16, ones16, deg_out, ones_v, accd))
    else:
        def body(y, idx2, zer64, agg_out,
                 idx_v, r0, r1, r2, r3, acc, g0, g1, g2, g3, s0, s1, s2, s3):
            _seg_body(y, idx2, zer64, agg_out, idx_v, [r0, r1, r2, r3], acc,
                      [g0, g1, g2, g3], [s0, s1, s2, s3], deg=None)

    return pl.kernel(body, out_type=tuple(outs) if with_deg else outs[0],
                     mesh=mesh, scratch_types=scratch,
                     compiler_params=pltpu.CompilerParams(
                         use_tc_tiling_on_sc=False))


@functools.cache
def _get_seg_sum(with_deg):
    return _make_seg_sum(with_deg)


def _seg_sum_deg(y, idx2, zer64, zer16, ones16):
    return _get_seg_sum(True)(y, idx2, zer64, zer16, ones16)


def _seg_sum(y, idx2, zer64):
    return _get_seg_sum(False)(y, idx2, zer64)


def _mm_dual_body(xl_ref, xp_ref, wl_ref, wp_ref, o_ref):
    sel = pl.program_id(0) < NS
    x = jnp.where(sel, xl_ref[...], xp_ref[...])
    w = jnp.where(sel, wl_ref[...], wp_ref[...])
    o_ref[...] = jnp.dot(x, w,
                         preferred_element_type=jnp.float32
                         ).astype(jnp.bfloat16)


def _mm_dual(xl, xp, wl, wp):
    blk = TOT // 32
    half = lambda i: jnp.where(i < NS, i, i - NS)
    return pl.pallas_call(
        _mm_dual_body,
        grid=(32,),
        in_specs=[
            pl.BlockSpec((blk, DIN), lambda i: (half(i), 0)),
            pl.BlockSpec((blk, DIN), lambda i: (half(i), 0)),
            pl.BlockSpec((DIN, EMB), lambda i: (0, 0)),
            pl.BlockSpec((DIN, EMB), lambda i: (0, 0)),
        ],
        out_specs=pl.BlockSpec((blk, EMB), lambda i: (i, 0)),
        out_shape=jax.ShapeDtypeStruct((TOT, EMB), jnp.bfloat16),
    )(xl, xp, wl, wp)


def _combine_body(parts_ref, deg_ref, bli_ref, bpi_ref, wl_ref, wp_ref,
                  blo_ref, bpo_ref, o_ref):
    sel = pl.program_id(0) < NS
    p = (parts_ref[0].astype(jnp.float32) + parts_ref[1].astype(jnp.float32))
    deg = deg_ref[0, :, 0:1] + deg_ref[1, :, 0:1]
    b_in = jnp.where(sel, bli_ref[...], bpi_ref[...])
    w = jnp.where(sel, wl_ref[...], wp_ref[...])
    b_out = jnp.where(sel, blo_ref[...], bpo_ref[...])
    h = jnp.maximum(p / jnp.maximum(deg, 1.0) + b_in, 0.0)
    o_ref[...] = (jnp.dot(h, w, preferred_element_type=jnp.float32)
                  + b_out).astype(o_ref.dtype)


def _combine(parts, degp, bl_in, bp_in, wl, wp, bl_out, bp_out, out_dtype):
    blk = TOT // 32
    b2 = lambda b: b.reshape(1, EMB)
    return pl.pallas_call(
        _combine_body,
        grid=(32,),
        in_specs=[
            pl.BlockSpec((NC, blk, EMB), lambda i: (0, i, 0)),
            pl.BlockSpec((NC, blk, DEG_W), lambda i: (0, i, 0)),
            pl.BlockSpec((1, EMB), lambda i: (0, 0)),
            pl.BlockSpec((1, EMB), lambda i: (0, 0)),
            pl.BlockSpec((EMB, EMB), lambda i: (0, 0)),
            pl.BlockSpec((EMB, EMB), lambda i: (0, 0)),
            pl.BlockSpec((1, EMB), lambda i: (0, 0)),
            pl.BlockSpec((1, EMB), lambda i: (0, 0)),
        ],
        out_specs=pl.BlockSpec((blk, EMB), lambda i: (i, 0)),
        out_shape=jax.ShapeDtypeStruct((TOT, EMB), out_dtype),
    )(parts, degp, b2(bl_in), b2(bp_in), wl, wp, b2(bl_out), b2(bp_out))


def _scores_body(a_ref, b_ref, o_ref):
    s = lax.dot_general(a_ref[...], b_ref[...], (((1,), (1,)), ((), ())),
                        preferred_element_type=jnp.float32)
    o_ref[...] = 1.0 / (1.0 + jnp.exp(-s * SCALE))


def _scores(ab):
    bm, bn = 1000, 1024
    return pl.pallas_call(
        _scores_body,
        grid=(N // bm, -(-N // bn)),
        in_specs=[
            pl.BlockSpec((bm, EMB), lambda i, j: (i, 0)),
            pl.BlockSpec((bn, EMB), lambda i, j: (j + REG // bn, 0)),
        ],
        out_specs=pl.BlockSpec((bm, bn), lambda i, j: (i, j)),
        out_shape=jax.ShapeDtypeStruct((N, N), jnp.float32),
    )(ab, ab)


def kernel(logic_x, phys_x, Wl1, bl1, Wl2, bl2, Wp1, bp1, Wp2, bp2,
           Was, bas, Wap, bap, logic_edge_index, phys_edge_index):
    f32 = jnp.float32
    src = jnp.concatenate([
        logic_edge_index[0].astype(jnp.int32),
        phys_edge_index[0].astype(jnp.int32) + REG,
        jnp.zeros((E_PAD - E_TOT,), jnp.int32),
    ])
    dst = jnp.concatenate([
        logic_edge_index[1].astype(jnp.int32),
        phys_edge_index[1].astype(jnp.int32) + REG,
        N + (jnp.arange(E_PAD - E_TOT, dtype=jnp.int32) % (REG - N)),
    ])
    nw = NC * NS
    idx2 = jnp.stack([src.reshape(CHUNKS_PER_TILE, nw, LANE),
                      dst.reshape(CHUNKS_PER_TILE, nw, LANE)],
                     axis=2).transpose(1, 0, 2, 3).reshape(
                         nw * CHUNKS_PER_TILE * 2, LANE)
    zer64 = jnp.zeros((ROWS_PER_TILE, EMB), jnp.bfloat16)
    zer16 = jnp.zeros((ROWS_PER_TILE, DEG_W), f32)
    ones16 = jnp.ones((LANE, DEG_W), f32)
    zb = jnp.zeros((EMB,), f32)

    y1 = _mm_dual(logic_x, phys_x, Wl1, Wp1)
    agg1, degp = _seg_sum_deg(y1, idx2, zer64, zer16, ones16)
    y2 = _combine(agg1, degp, bl1, bp1, Wl2, Wp2, zb, zb, jnp.bfloat16)
    agg2 = _seg_sum(y2, idx2, zer64)
    ab = _combine(agg2, degp, bl2, bp2, Was, Wap, bas, bap, jnp.float32)
    return _scores(ab)


# speedup vs baseline: 1.5273x; 1.0144x over previous
import functools

import jax
import jax.numpy as jnp
from jax import lax
from jax.experimental import pallas as pl
from jax.experimental.pallas import tpu as pltpu
from jax.experimental.pallas import tpu_sc as plsc

N = 10000
DIN = 128
EMB = 64
REG = 10240
TOT = 2 * REG
NC, NS = 2, 16
ROWS_PER_TILE = TOT // NS
LANE = 128
E_TOT = 320000 + 160000
CHUNKS_PER_TILE = 120
QC = CHUNKS_PER_TILE
TOT_CHUNKS = NC * NS * CHUNKS_PER_TILE
E_PAD = TOT_CHUNKS * LANE
DEG_W = 16
NBUF = 8
SCALE = EMB ** (-0.5)


def _seg_body(y, idx2, zer64, agg_out, idx_v, rows, acc, semg, sems,
              deg=None):
    cid = lax.axis_index("c")
    sid = lax.axis_index("s")
    w = cid * NS + sid
    c2 = 2 * CHUNKS_PER_TILE
    C = CHUNKS_PER_TILE
    row0 = sid * ROWS_PER_TILE
    pltpu.sync_copy(zer64, acc.at[pl.ds(row0, ROWS_PER_TILE)])
    if deg is not None:
        zer16, ones16, deg_out, ones_v, accd = deg
        pltpu.sync_copy(zer16, accd.at[pl.ds(row0, ROWS_PER_TILE)])
        pltpu.sync_copy(ones16, ones_v)
    pltpu.sync_copy(idx2.at[pl.ds(w * c2, c2)], idx_v)
    plsc.subcore_barrier()

    def step(j, b, wait_scat, start_gather):
        bs = (b + NBUF - 2) % NBUF
        if wait_scat:
            pltpu.make_async_copy(rows[bs], acc.at[idx_v.at[(j - 2) * 2 + 1]],
                                  sems[bs]).wait()
            if deg is not None:
                pltpu.make_async_copy(ones_v,
                                      accd.at[idx_v.at[(j - 2) * 2 + 1]],
                                      sems[bs]).wait()
        if start_gather:
            pltpu.async_copy(y.at[idx_v.at[(j + NBUF - 2) * 2]], rows[bs],
                             semg[bs])
        pltpu.make_async_copy(y.at[idx_v.at[j * 2]], rows[b], semg[b]).wait()
        pltpu.async_copy(rows[b], acc.at[idx_v.at[j * 2 + 1]], sems[b],
                         add=True)
        if deg is not None:
            pltpu.async_copy(ones_v, accd.at[idx_v.at[j * 2 + 1]], sems[b],
                             add=True)

    for b in range(NBUF - 2):
        pltpu.async_copy(y.at[idx_v.at[2 * b]], rows[b], semg[b])
    for j in range(NBUF):
        step(j, j, j >= 2, True)

    def bodyn(jj, carry):
        for u in range(NBUF):
            step(jj * NBUF + u, u, True, True)
        return carry

    lax.fori_loop(1, C // NBUF - 1, bodyn, 0)
    for j in range(C - NBUF, C):
        step(j, j % NBUF, True, j < C - (NBUF - 2))
    for j in (C - 2, C - 1):
        b = j % NBUF
        pltpu.make_async_copy(rows[b], acc.at[idx_v.at[j * 2 + 1]],
                              sems[b]).wait()
        if deg is not None:
            pltpu.make_async_copy(ones_v, accd.at[idx_v.at[j * 2 + 1]],
                                  sems[b]).wait()
    plsc.subcore_barrier()
    pltpu.sync_copy(acc.at[pl.ds(row0, ROWS_PER_TILE)],
                    agg_out.at[cid, pl.ds(row0, ROWS_PER_TILE)])
    if deg is not None:
        pltpu.sync_copy(accd.at[pl.ds(row0, ROWS_PER_TILE)],
                        deg_out.at[cid, pl.ds(row0, ROWS_PER_TILE)])


def _make_seg_sum(with_deg):
    mesh = plsc.VectorSubcoreMesh(core_axis_name="c", subcore_axis_name="s")
    outs = [jax.ShapeDtypeStruct((NC, TOT, EMB), jnp.bfloat16)]
    scratch = (
        [pltpu.VMEM((2 * QC, LANE), jnp.int32)]
        + [pltpu.VMEM((LANE, EMB), jnp.bfloat16)] * NBUF
        + [pltpu.VMEM_SHARED((TOT, EMB), jnp.bfloat16)]
        + [pltpu.SemaphoreType.DMA] * (2 * NBUF)
    )
    if with_deg:
        outs.append(jax.ShapeDtypeStruct((NC, TOT, DEG_W), jnp.float32))
        scratch += [
            pltpu.VMEM((LANE, DEG_W), jnp.float32),
            pltpu.VMEM_SHARED((TOT, DEG_W), jnp.float32),
        ]

        def body(y, idx2, zer64, zer16, ones16, agg_out, deg_out, *s):
            idx_v, rows = s[0], list(s[1:1 + NBUF])
            acc = s[1 + NBUF]
            semg = list(s[2 + NBUF:2 + 2 * NBUF])
            sems = list(s[2 + 2 * NBUF:2 + 3 * NBUF])
            ones_v, accd = s[2 + 3 * NBUF], s[3 + 3 * NBUF]
            _seg_body(y, idx2, zer64, agg_out, idx_v, rows, acc, semg, sems,
                      deg=(zer16, ones16, deg_out, ones_v, accd))
    else:
        def body(y, idx2, zer64, agg_out, *s):
            idx_v, rows = s[0], list(s[1:1 + NBUF])
            acc = s[1 + NBUF]
            semg = list(s[2 + NBUF:2 + 2 * NBUF])
            sems = list(s[2 + 2 * NBUF:2 + 3 * NBUF])
            _seg_body(y, idx2, zer64, agg_out, idx_v, rows, acc, semg, sems,
                      deg=None)

    return pl.kernel(body, out_type=tuple(outs) if with_deg else outs[0],
                     mesh=mesh, scratch_types=scratch,
                     compiler_params=pltpu.CompilerParams(
                         use_tc_tiling_on_sc=False))


@functools.cache
def _get_seg_sum(with_deg):
    return _make_seg_sum(with_deg)


def _seg_sum_deg(y, idx2, zer64, zer16, ones16):
    return _get_seg_sum(True)(y, idx2, zer64, zer16, ones16)


def _seg_sum(y, idx2, zer64):
    return _get_seg_sum(False)(y, idx2, zer64)


def _mm_dual_body(xl_ref, xp_ref, wl_ref, wp_ref, o_ref):
    sel = pl.program_id(0) < NS
    x = jnp.where(sel, xl_ref[...], xp_ref[...])
    w = jnp.where(sel, wl_ref[...], wp_ref[...])
    o_ref[...] = jnp.dot(x, w,
                         preferred_element_type=jnp.float32
                         ).astype(jnp.bfloat16)


def _mm_dual(xl, xp, wl, wp):
    blk = TOT // 32
    half = lambda i: jnp.where(i < NS, i, i - NS)
    return pl.pallas_call(
        _mm_dual_body,
        grid=(32,),
        in_specs=[
            pl.BlockSpec((blk, DIN), lambda i: (half(i), 0)),
            pl.BlockSpec((blk, DIN), lambda i: (half(i), 0)),
            pl.BlockSpec((DIN, EMB), lambda i: (0, 0)),
            pl.BlockSpec((DIN, EMB), lambda i: (0, 0)),
        ],
        out_specs=pl.BlockSpec((blk, EMB), lambda i: (i, 0)),
        out_shape=jax.ShapeDtypeStruct((TOT, EMB), jnp.bfloat16),
    )(xl, xp, wl, wp)


def _combine_body(parts_ref, deg_ref, bli_ref, bpi_ref, wl_ref, wp_ref,
                  blo_ref, bpo_ref, o_ref):
    sel = pl.program_id(0) < NS
    p = (parts_ref[0].astype(jnp.float32) + parts_ref[1].astype(jnp.float32))
    deg = deg_ref[0, :, 0:1] + deg_ref[1, :, 0:1]
    b_in = jnp.where(sel, bli_ref[...], bpi_ref[...])
    w = jnp.where(sel, wl_ref[...], wp_ref[...])
    b_out = jnp.where(sel, blo_ref[...], bpo_ref[...])
    h = jnp.maximum(p / jnp.maximum(deg, 1.0) + b_in, 0.0)
    o_ref[...] = (jnp.dot(h, w, preferred_element_type=jnp.float32)
                  + b_out).astype(o_ref.dtype)


def _combine(parts, degp, bl_in, bp_in, wl, wp, bl_out, bp_out, out_dtype):
    blk = TOT // 32
    b2 = lambda b: b.reshape(1, EMB)
    return pl.pallas_call(
        _combine_body,
        grid=(32,),
        in_specs=[
            pl.BlockSpec((NC, blk, EMB), lambda i: (0, i, 0)),
            pl.BlockSpec((NC, blk, DEG_W), lambda i: (0, i, 0)),
            pl.BlockSpec((1, EMB), lambda i: (0, 0)),
            pl.BlockSpec((1, EMB), lambda i: (0, 0)),
            pl.BlockSpec((EMB, EMB), lambda i: (0, 0)),
            pl.BlockSpec((EMB, EMB), lambda i: (0, 0)),
            pl.BlockSpec((1, EMB), lambda i: (0, 0)),
            pl.BlockSpec((1, EMB), lambda i: (0, 0)),
        ],
        out_specs=pl.BlockSpec((blk, EMB), lambda i: (i, 0)),
        out_shape=jax.ShapeDtypeStruct((TOT, EMB), out_dtype),
    )(parts, degp, b2(bl_in), b2(bp_in), wl, wp, b2(bl_out), b2(bp_out))


def _scores_body(a_ref, b_ref, o_ref):
    s = lax.dot_general(a_ref[...], b_ref[...], (((1,), (1,)), ((), ())),
                        preferred_element_type=jnp.float32)
    o_ref[...] = 1.0 / (1.0 + jnp.exp(-s * SCALE))


def _scores(ab):
    bm, bn = 1000, 1024
    return pl.pallas_call(
        _scores_body,
        grid=(N // bm, -(-N // bn)),
        in_specs=[
            pl.BlockSpec((bm, EMB), lambda i, j: (i, 0)),
            pl.BlockSpec((bn, EMB), lambda i, j: (j + REG // bn, 0)),
        ],
        out_specs=pl.BlockSpec((bm, bn), lambda i, j: (i, j)),
        out_shape=jax.ShapeDtypeStruct((N, N), jnp.float32),
    )(ab, ab)


def kernel(logic_x, phys_x, Wl1, bl1, Wl2, bl2, Wp1, bp1, Wp2, bp2,
           Was, bas, Wap, bap, logic_edge_index, phys_edge_index):
    f32 = jnp.float32
    src = jnp.concatenate([
        logic_edge_index[0].astype(jnp.int32),
        phys_edge_index[0].astype(jnp.int32) + REG,
        jnp.zeros((E_PAD - E_TOT,), jnp.int32),
    ])
    dst = jnp.concatenate([
        logic_edge_index[1].astype(jnp.int32),
        phys_edge_index[1].astype(jnp.int32) + REG,
        N + (jnp.arange(E_PAD - E_TOT, dtype=jnp.int32) % (REG - N)),
    ])
    nw = NC * NS
    idx2 = jnp.stack([src.reshape(CHUNKS_PER_TILE, nw, LANE),
                      dst.reshape(CHUNKS_PER_TILE, nw, LANE)],
                     axis=2).transpose(1, 0, 2, 3).reshape(
                         nw * CHUNKS_PER_TILE * 2, LANE)
    zer64 = jnp.zeros((ROWS_PER_TILE, EMB), jnp.bfloat16)
    zer16 = jnp.zeros((ROWS_PER_TILE, DEG_W), f32)
    ones16 = jnp.ones((LANE, DEG_W), f32)
    zb = jnp.zeros((EMB,), f32)

    y1 = _mm_dual(logic_x, phys_x, Wl1, Wp1)
    agg1, degp = _seg_sum_deg(y1, idx2, zer64, zer16, ones16)
    y2 = _combine(agg1, degp, bl1, bp1, Wl2, Wp2, zb, zb, jnp.bfloat16)
    agg2 = _seg_sum(y2, idx2, zer64)
    ab = _combine(agg2, degp, bl2, bp2, Was, Wap, bas, bap, jnp.float32)
    return _scores(ab)


# speedup vs baseline: 1.6136x; 1.0565x over previous
import functools

import jax
import jax.numpy as jnp
from jax import lax
from jax.experimental import pallas as pl
from jax.experimental.pallas import tpu as pltpu
from jax.experimental.pallas import tpu_sc as plsc

N = 10000
DIN = 128
EMB = 64
REG = 10240
TOT = 2 * REG
NC, NS = 2, 16
ROWS_PER_TILE = TOT // NS
LANE = 128
E_TOT = 320000 + 160000
CHUNKS_PER_TILE = 120
QC = CHUNKS_PER_TILE
TOT_CHUNKS = NC * NS * CHUNKS_PER_TILE
E_PAD = TOT_CHUNKS * LANE
DEG_W = 16
NBUF = 8
SCALE = EMB ** (-0.5)


def _seg_body(y, idx2, zer64, agg_out, idx_v, rows, acc, semg, sems,
              deg=None):
    cid = lax.axis_index("c")
    sid = lax.axis_index("s")
    w = cid * NS + sid
    c2 = 2 * CHUNKS_PER_TILE
    C = CHUNKS_PER_TILE
    row0 = sid * ROWS_PER_TILE
    pltpu.sync_copy(zer64, acc.at[pl.ds(row0, ROWS_PER_TILE)])
    if deg is not None:
        zer16, ones16, deg_out, ones_v, accd = deg
        pltpu.sync_copy(zer16, accd.at[pl.ds(row0, ROWS_PER_TILE)])
        pltpu.sync_copy(ones16, ones_v)
    pltpu.sync_copy(idx2.at[pl.ds(w * c2, c2)], idx_v)
    plsc.subcore_barrier()

    def step(j, b, wait_scat, start_gather):
        bs = (b + NBUF - 2) % NBUF
        if wait_scat:
            pltpu.make_async_copy(rows[bs], acc.at[idx_v.at[(j - 2) * 2 + 1]],
                                  sems[bs]).wait()
            if deg is not None:
                pltpu.make_async_copy(ones_v,
                                      accd.at[idx_v.at[(j - 2) * 2 + 1]],
                                      sems[bs]).wait()
        if start_gather:
            pltpu.async_copy(y.at[idx_v.at[(j + NBUF - 2) * 2]], rows[bs],
                             semg[bs])
        pltpu.make_async_copy(y.at[idx_v.at[j * 2]], rows[b], semg[b]).wait()
        pltpu.async_copy(rows[b], acc.at[idx_v.at[j * 2 + 1]], sems[b],
                         add=True)
        if deg is not None:
            pltpu.async_copy(ones_v, accd.at[idx_v.at[j * 2 + 1]], sems[b],
                             add=True)

    for b in range(NBUF - 2):
        pltpu.async_copy(y.at[idx_v.at[2 * b]], rows[b], semg[b])
    for j in range(NBUF):
        step(j, j, j >= 2, True)

    def bodyn(jj, carry):
        for u in range(NBUF):
            step(jj * NBUF + u, u, True, True)
        return carry

    lax.fori_loop(1, C // NBUF - 1, bodyn, 0)
    for j in range(C - NBUF, C):
        step(j, j % NBUF, True, j < C - (NBUF - 2))
    for j in (C - 2, C - 1):
        b = j % NBUF
        pltpu.make_async_copy(rows[b], acc.at[idx_v.at[j * 2 + 1]],
                              sems[b]).wait()
        if deg is not None:
            pltpu.make_async_copy(ones_v, accd.at[idx_v.at[j * 2 + 1]],
                                  sems[b]).wait()
    plsc.subcore_barrier()
    pltpu.sync_copy(acc.at[pl.ds(row0, ROWS_PER_TILE)],
                    agg_out.at[cid, pl.ds(row0, ROWS_PER_TILE)])
    if deg is not None:
        pltpu.sync_copy(accd.at[pl.ds(row0, ROWS_PER_TILE)],
                        deg_out.at[cid, pl.ds(row0, ROWS_PER_TILE)])


def _make_seg_sum(with_deg):
    mesh = plsc.VectorSubcoreMesh(core_axis_name="c", subcore_axis_name="s")
    outs = [jax.ShapeDtypeStruct((NC, TOT, EMB), jnp.bfloat16)]
    scratch = (
        [pltpu.VMEM((2 * QC, LANE), jnp.int32)]
        + [pltpu.VMEM((LANE, EMB), jnp.bfloat16)] * NBUF
        + [pltpu.VMEM_SHARED((TOT, EMB), jnp.bfloat16)]
        + [pltpu.SemaphoreType.DMA] * (2 * NBUF)
    )
    if with_deg:
        outs.append(jax.ShapeDtypeStruct((NC, TOT, DEG_W), jnp.float32))
        scratch += [
            pltpu.VMEM((LANE, DEG_W), jnp.float32),
            pltpu.VMEM_SHARED((TOT, DEG_W), jnp.float32),
        ]

        def body(y, idx2, zer64, zer16, ones16, agg_out, deg_out, *s):
            idx_v, rows = s[0], list(s[1:1 + NBUF])
            acc = s[1 + NBUF]
            semg = list(s[2 + NBUF:2 + 2 * NBUF])
            sems = list(s[2 + 2 * NBUF:2 + 3 * NBUF])
            ones_v, accd = s[2 + 3 * NBUF], s[3 + 3 * NBUF]
            _seg_body(y, idx2, zer64, agg_out, idx_v, rows, acc, semg, sems,
                      deg=(zer16, ones16, deg_out, ones_v, accd))
    else:
        def body(y, idx2, zer64, agg_out, *s):
            idx_v, rows = s[0], list(s[1:1 + NBUF])
            acc = s[1 + NBUF]
            semg = list(s[2 + NBUF:2 + 2 * NBUF])
            sems = list(s[2 + 2 * NBUF:2 + 3 * NBUF])
            _seg_body(y, idx2, zer64, agg_out, idx_v, rows, acc, semg, sems,
                      deg=None)

    return pl.kernel(body, out_type=tuple(outs) if with_deg else outs[0],
                     mesh=mesh, scratch_types=scratch,
                     compiler_params=pltpu.CompilerParams(
                         use_tc_tiling_on_sc=False))


@functools.cache
def _get_seg_sum(with_deg):
    return _make_seg_sum(with_deg)


def _seg_sum_deg(y, idx2, zer64, zer16, ones16):
    return _get_seg_sum(True)(y, idx2, zer64, zer16, ones16)


def _seg_sum(y, idx2, zer64):
    return _get_seg_sum(False)(y, idx2, zer64)


def _mm_dual_body(xl_ref, xp_ref, wl_ref, wp_ref, o_ref):
    sel = pl.program_id(0) < NS
    x = jnp.where(sel, xl_ref[...], xp_ref[...])
    w = jnp.where(sel, wl_ref[...], wp_ref[...])
    o_ref[...] = jnp.dot(x, w,
                         preferred_element_type=jnp.float32
                         ).astype(jnp.bfloat16)


def _mm_dual(xl, xp, wl, wp):
    blk = TOT // 32
    half = lambda i: jnp.where(i < NS, i, i - NS)
    return pl.pallas_call(
        _mm_dual_body,
        grid=(32,),
        in_specs=[
            pl.BlockSpec((blk, DIN), lambda i: (half(i), 0)),
            pl.BlockSpec((blk, DIN), lambda i: (half(i), 0)),
            pl.BlockSpec((DIN, EMB), lambda i: (0, 0)),
            pl.BlockSpec((DIN, EMB), lambda i: (0, 0)),
        ],
        out_specs=pl.BlockSpec((blk, EMB), lambda i: (i, 0)),
        out_shape=jax.ShapeDtypeStruct((TOT, EMB), jnp.bfloat16),
    )(xl, xp, wl, wp)


def _combine_body(parts_ref, deg_ref, bli_ref, bpi_ref, wl_ref, wp_ref,
                  blo_ref, bpo_ref, o_ref):
    sel = pl.program_id(0) < NS
    p = (parts_ref[0].astype(jnp.float32) + parts_ref[1].astype(jnp.float32))
    deg = deg_ref[0, :, 0:1] + deg_ref[1, :, 0:1]
    b_in = jnp.where(sel, bli_ref[...], bpi_ref[...])
    w = jnp.where(sel, wl_ref[...], wp_ref[...])
    b_out = jnp.where(sel, blo_ref[...], bpo_ref[...])
    h = jnp.maximum(p / jnp.maximum(deg, 1.0) + b_in, 0.0)
    o_ref[...] = (jnp.dot(h, w, preferred_element_type=jnp.float32)
                  + b_out).astype(o_ref.dtype)


def _combine(parts, degp, bl_in, bp_in, wl, wp, bl_out, bp_out, out_dtype):
    blk = TOT // 32
    b2 = lambda b: b.reshape(1, EMB)
    return pl.pallas_call(
        _combine_body,
        grid=(32,),
        in_specs=[
            pl.BlockSpec((NC, blk, EMB), lambda i: (0, i, 0)),
            pl.BlockSpec((NC, blk, DEG_W), lambda i: (0, i, 0)),
            pl.BlockSpec((1, EMB), lambda i: (0, 0)),
            pl.BlockSpec((1, EMB), lambda i: (0, 0)),
            pl.BlockSpec((EMB, EMB), lambda i: (0, 0)),
            pl.BlockSpec((EMB, EMB), lambda i: (0, 0)),
            pl.BlockSpec((1, EMB), lambda i: (0, 0)),
            pl.BlockSpec((1, EMB), lambda i: (0, 0)),
        ],
        out_specs=pl.BlockSpec((blk, EMB), lambda i: (i, 0)),
        out_shape=jax.ShapeDtypeStruct((TOT, EMB), out_dtype),
    )(parts, degp, b2(bl_in), b2(bp_in), wl, wp, b2(bl_out), b2(bp_out))


def _scores_body(a_ref, b_ref, o_ref):
    s = lax.dot_general(a_ref[...], b_ref[...], (((1,), (1,)), ((), ())),
                        preferred_element_type=jnp.float32)
    o_ref[...] = 1.0 / (1.0 + jnp.exp(-s[:, :N] * SCALE))


def _scores(ab):
    bm = 400
    return pl.pallas_call(
        _scores_body,
        grid=(N // bm,),
        in_specs=[
            pl.BlockSpec((bm, EMB), lambda i: (i, 0)),
            pl.BlockSpec((REG, EMB), lambda i: (1, 0)),
        ],
        out_specs=pl.BlockSpec((bm, N), lambda i: (i, 0)),
        out_shape=jax.ShapeDtypeStruct((N, N), jnp.float32),
    )(ab, ab)


def kernel(logic_x, phys_x, Wl1, bl1, Wl2, bl2, Wp1, bp1, Wp2, bp2,
           Was, bas, Wap, bap, logic_edge_index, phys_edge_index):
    f32 = jnp.float32
    src = jnp.concatenate([
        logic_edge_index[0].astype(jnp.int32),
        phys_edge_index[0].astype(jnp.int32) + REG,
        jnp.zeros((E_PAD - E_TOT,), jnp.int32),
    ])
    dst = jnp.concatenate([
        logic_edge_index[1].astype(jnp.int32),
        phys_edge_index[1].astype(jnp.int32) + REG,
        N + (jnp.arange(E_PAD - E_TOT, dtype=jnp.int32) % (REG - N)),
    ])
    nw = NC * NS
    idx2 = jnp.stack([src.reshape(CHUNKS_PER_TILE, nw, LANE),
                      dst.reshape(CHUNKS_PER_TILE, nw, LANE)],
                     axis=2).transpose(1, 0, 2, 3).reshape(
                         nw * CHUNKS_PER_TILE * 2, LANE)
    zer64 = jnp.zeros((ROWS_PER_TILE, EMB), jnp.bfloat16)
    zer16 = jnp.zeros((ROWS_PER_TILE, DEG_W), f32)
    ones16 = jnp.ones((LANE, DEG_W), f32)
    zb = jnp.zeros((EMB,), f32)

    y1 = _mm_dual(logic_x, phys_x, Wl1, Wp1)
    agg1, degp = _seg_sum_deg(y1, idx2, zer64, zer16, ones16)
    y2 = _combine(agg1, degp, bl1, bp1, Wl2, Wp2, zb, zb, jnp.bfloat16)
    agg2 = _seg_sum(y2, idx2, zer64)
    ab = _combine(agg2, degp, bl2, bp2, Was, Wap, bas, bap, jnp.float32)
    return _scores(ab)
